# Initial kernel scaffold; baseline (speedup 1.0000x reference)
import jax
import jax.numpy as jnp
from jax.experimental import pallas as pl


def kernel(x, edge_index, edge_attr, batch, Wp, bp, W0, Asrc0, Adst0, Aedge0, We0, b0, g0, beta0, W1, Asrc1, Adst1, Aedge1, We1, b1, g1, beta1, W2, Asrc2, Adst2, Aedge2, We2, b2, g2, beta2, Wc1, bc1, Wc2, bc2, Wn1, bn1, Wn2, bn2):
    raise NotImplementedError("write your pallas kernel here")


# speedup vs baseline: 8.9823x; 8.9823x over previous
import jax
import jax.numpy as jnp
from jax.experimental import pallas as pl

_NBLK = 2000
_EBLK = 4000
_NGRAPHS = 16


def _expand_a(a):
    heads, ch = a.shape
    return (a[:, :, None] * jnp.eye(heads, dtype=a.dtype)[:, None, :]).reshape(
        heads * ch, heads)


def _proj_kernel(x_ref, wp_ref, bp_ref, o_ref):
    o_ref[...] = jax.nn.relu(
        jnp.dot(x_ref[...], wp_ref[...], preferred_element_type=jnp.float32)
        + bp_ref[...])


def _node_mm_kernel(h_ref, w_ref, asrc_ref, adst_ref, hh_ref, as_ref, ad_ref):
    hh = jnp.dot(h_ref[...], w_ref[...], preferred_element_type=jnp.float32)
    hh_ref[...] = hh
    as_ref[...] = jnp.dot(hh, asrc_ref[...], preferred_element_type=jnp.float32)
    ad_ref[...] = jnp.dot(hh, adst_ref[...], preferred_element_type=jnp.float32)


def _alpha_kernel(ea_ref, we_ref, ae_ref, asg_ref, adg_ref, o_ref):
    ee = jnp.dot(ea_ref[...], we_ref[...], preferred_element_type=jnp.float32)
    a_edge = jnp.dot(ee, ae_ref[...], preferred_element_type=jnp.float32)
    a = asg_ref[...] + adg_ref[...] + a_edge
    o_ref[...] = jnp.where(a >= 0, a, 0.2 * a)


def _ex_kernel(alpha_ref, amax_ref, o_ref):
    o_ref[...] = jnp.exp(alpha_ref[...] - amax_ref[...])


def _msg_kernel(ex_ref, den_ref, hh_ref, r_ref, o_ref):
    att = ex_ref[...] / (den_ref[...] + 1e-16)
    att64 = jnp.dot(att, r_ref[...], preferred_element_type=jnp.float32)
    o_ref[...] = hh_ref[...] * att64


def _post_kernel(agg_ref, hres_ref, b_ref, g_ref, beta_ref, o_ref):
    z = agg_ref[...] + b_ref[...] + hres_ref[...]
    mu = jnp.mean(z, axis=-1, keepdims=True)
    zc = z - mu
    var = jnp.mean(zc * zc, axis=-1, keepdims=True)
    hn = zc * jax.lax.rsqrt(var + 1e-5) * g_ref[...] + beta_ref[...]
    o_ref[...] = jax.nn.relu(hn)


def _pool_kernel(oh_ref, h_ref, sum_ref, cnt_ref):
    @pl.when(pl.program_id(0) == 0)
    def _init():
        sum_ref[...] = jnp.zeros_like(sum_ref)
        cnt_ref[...] = jnp.zeros_like(cnt_ref)

    oh = oh_ref[...]
    h = h_ref[...]
    sum_ref[...] += jax.lax.dot_general(
        oh, h, (((0,), (0,)), ((), ())), preferred_element_type=jnp.float32)
    cnt_ref[...] += jax.lax.dot_general(
        oh, jnp.ones_like(h), (((0,), (0,)), ((), ())),
        preferred_element_type=jnp.float32)


def _head_kernel(sum_ref, cnt_ref, wc1_ref, bc1_ref, wc2_ref, bc2_ref,
                 wn1_ref, bn1_ref, wn2_ref, bn2_ref,
                 gemb_ref, scene_ref, nav_ref):
    gemb = sum_ref[...] / jnp.maximum(cnt_ref[...], 1.0)
    gemb_ref[...] = gemb
    hidc = jax.nn.relu(
        jnp.dot(gemb, wc1_ref[...], preferred_element_type=jnp.float32)
        + bc1_ref[...])
    scene_ref[...] = (
        jnp.dot(hidc, wc2_ref[...], preferred_element_type=jnp.float32)
        + bc2_ref[...])
    hidn = jax.nn.relu(
        jnp.dot(gemb, wn1_ref[...], preferred_element_type=jnp.float32)
        + bn1_ref[...])
    nav_ref[...] = jax.nn.sigmoid(
        jnp.dot(hidn, wn2_ref[...], preferred_element_type=jnp.float32)
        + bn2_ref[...])


def _full(shape):
    return pl.BlockSpec(shape, lambda *i: tuple(0 for _ in shape))


def kernel(x, edge_index, edge_attr, batch, Wp, bp,
           W0, Asrc0, Adst0, Aedge0, We0, b0, g0, beta0,
           W1, Asrc1, Adst1, Aedge1, We1, b1, g1, beta1,
           W2, Asrc2, Adst2, Aedge2, We2, b2, g2, beta2,
           Wc1, bc1, Wc2, bc2, Wn1, bn1, Wn2, bn2):
    n = x.shape[0]
    e = edge_attr.shape[0]
    hid = Wp.shape[1]
    heads, ch = Asrc0.shape

    src0 = edge_index[0]
    dst0 = edge_index[1]
    loop = jnp.arange(n, dtype=src0.dtype)
    src = jnp.concatenate([src0, loop])
    dst = jnp.concatenate([dst0, loop])
    ea = jnp.concatenate([
        edge_attr,
        jnp.broadcast_to(jnp.mean(edge_attr, axis=0, keepdims=True),
                         (n, edge_attr.shape[1]))], axis=0)
    ee_rows = e + n

    gn = pl.cdiv(n, _NBLK)
    ge = pl.cdiv(ee_rows, _EBLK)

    rmat = jnp.repeat(jnp.eye(heads, dtype=jnp.float32), ch, axis=1)

    h = pl.pallas_call(
        _proj_kernel,
        grid=(gn,),
        in_specs=[pl.BlockSpec((_NBLK, x.shape[1]), lambda i: (i, 0)),
                  _full(Wp.shape),
                  _full((1, hid))],
        out_specs=pl.BlockSpec((_NBLK, hid), lambda i: (i, 0)),
        out_shape=jax.ShapeDtypeStruct((n, hid), jnp.float32),
    )(x, Wp, bp.reshape(1, hid))

    layers = [
        (W0, Asrc0, Adst0, Aedge0, We0, b0, g0, beta0),
        (W1, Asrc1, Adst1, Aedge1, We1, b1, g1, beta1),
        (W2, Asrc2, Adst2, Aedge2, We2, b2, g2, beta2),
    ]

    for (W, Asrc, Adst, Aedge, We, b, g, beta) in layers:
        hres = h
        hh, a_src, a_dst = pl.pallas_call(
            _node_mm_kernel,
            grid=(gn,),
            in_specs=[pl.BlockSpec((_NBLK, hid), lambda i: (i, 0)),
                      _full(W.shape),
                      _full((hid, heads)),
                      _full((hid, heads))],
            out_specs=[pl.BlockSpec((_NBLK, hid), lambda i: (i, 0)),
                       pl.BlockSpec((_NBLK, heads), lambda i: (i, 0)),
                       pl.BlockSpec((_NBLK, heads), lambda i: (i, 0))],
            out_shape=[jax.ShapeDtypeStruct((n, hid), jnp.float32),
                       jax.ShapeDtypeStruct((n, heads), jnp.float32),
                       jax.ShapeDtypeStruct((n, heads), jnp.float32)],
        )(h, W, _expand_a(Asrc), _expand_a(Adst))

        asg = a_src[src]
        adg = a_dst[dst]

        alpha = pl.pallas_call(
            _alpha_kernel,
            grid=(ge,),
            in_specs=[pl.BlockSpec((_EBLK, ea.shape[1]), lambda i: (i, 0)),
                      _full(We.shape),
                      _full((hid, heads)),
                      pl.BlockSpec((_EBLK, heads), lambda i: (i, 0)),
                      pl.BlockSpec((_EBLK, heads), lambda i: (i, 0))],
            out_specs=pl.BlockSpec((_EBLK, heads), lambda i: (i, 0)),
            out_shape=jax.ShapeDtypeStruct((ee_rows, heads), jnp.float32),
        )(ea, We, _expand_a(Aedge), asg, adg)

        amax = jax.ops.segment_max(alpha, dst, num_segments=n)
        ex = pl.pallas_call(
            _ex_kernel,
            grid=(ge,),
            in_specs=[pl.BlockSpec((_EBLK, heads), lambda i: (i, 0)),
                      pl.BlockSpec((_EBLK, heads), lambda i: (i, 0))],
            out_specs=pl.BlockSpec((_EBLK, heads), lambda i: (i, 0)),
            out_shape=jax.ShapeDtypeStruct((ee_rows, heads), jnp.float32),
        )(alpha, amax[dst])

        den = jax.ops.segment_sum(ex, dst, num_segments=n)
        msg = pl.pallas_call(
            _msg_kernel,
            grid=(ge,),
            in_specs=[pl.BlockSpec((_EBLK, heads), lambda i: (i, 0)),
                      pl.BlockSpec((_EBLK, heads), lambda i: (i, 0)),
                      pl.BlockSpec((_EBLK, hid), lambda i: (i, 0)),
                      _full((heads, hid))],
            out_specs=pl.BlockSpec((_EBLK, hid), lambda i: (i, 0)),
            out_shape=jax.ShapeDtypeStruct((ee_rows, hid), jnp.float32),
        )(ex, den[dst], hh[src], rmat)

        agg = jax.ops.segment_sum(msg, dst, num_segments=n)

        h = pl.pallas_call(
            _post_kernel,
            grid=(gn,),
            in_specs=[pl.BlockSpec((_NBLK, hid), lambda i: (i, 0)),
                      pl.BlockSpec((_NBLK, hid), lambda i: (i, 0)),
                      _full((1, hid)),
                      _full((1, hid)),
                      _full((1, hid))],
            out_specs=pl.BlockSpec((_NBLK, hid), lambda i: (i, 0)),
            out_shape=jax.ShapeDtypeStruct((n, hid), jnp.float32),
        )(agg, hres, b.reshape(1, hid), g.reshape(1, hid),
          beta.reshape(1, hid))

    onehot = (batch[:, None] == jnp.arange(_NGRAPHS, dtype=batch.dtype)[None, :]
              ).astype(jnp.float32)
    sums, cnts = pl.pallas_call(
        _pool_kernel,
        grid=(gn,),
        in_specs=[pl.BlockSpec((_NBLK, _NGRAPHS), lambda i: (i, 0)),
                  pl.BlockSpec((_NBLK, hid), lambda i: (i, 0))],
        out_specs=[_full((_NGRAPHS, hid)), _full((_NGRAPHS, hid))],
        out_shape=[jax.ShapeDtypeStruct((_NGRAPHS, hid), jnp.float32),
                   jax.ShapeDtypeStruct((_NGRAPHS, hid), jnp.float32)],
    )(onehot, h)

    gemb, scene_logits, nav = pl.pallas_call(
        _head_kernel,
        in_specs=[_full((_NGRAPHS, hid)), _full((_NGRAPHS, hid)),
                  _full(Wc1.shape), _full((1, Wc1.shape[1])),
                  _full(Wc2.shape), _full((1, Wc2.shape[1])),
                  _full(Wn1.shape), _full((1, Wn1.shape[1])),
                  _full(Wn2.shape), _full((1, Wn2.shape[1]))],
        out_specs=[_full((_NGRAPHS, hid)),
                   _full((_NGRAPHS, Wc2.shape[1])),
                   _full((_NGRAPHS, Wn2.shape[1]))],
        out_shape=[jax.ShapeDtypeStruct((_NGRAPHS, hid), jnp.float32),
                   jax.ShapeDtypeStruct((_NGRAPHS, Wc2.shape[1]), jnp.float32),
                   jax.ShapeDtypeStruct((_NGRAPHS, Wn2.shape[1]), jnp.float32)],
    )(sums, cnts, Wc1, bc1.reshape(1, -1), Wc2, bc2.reshape(1, -1),
      Wn1, bn1.reshape(1, -1), Wn2, bn2.reshape(1, -1))

    return (h, scene_logits, nav, gemb)


# speedup vs baseline: 14.9586x; 1.6653x over previous
import jax
import jax.numpy as jnp
from jax.experimental import pallas as pl

_NBLK = 2000
_EBLK = 4000
_NGRAPHS = 16


def _expand_a(a):
    heads, ch = a.shape
    return (a[:, :, None] * jnp.eye(heads, dtype=a.dtype)[:, None, :]).reshape(
        heads * ch, heads)


def _proj_kernel(x_ref, wp_ref, bp_ref, o_ref):
    o_ref[...] = jax.nn.relu(
        jnp.dot(x_ref[...], wp_ref[...], preferred_element_type=jnp.float32)
        + bp_ref[...])


def _node_mm_kernel(h_ref, w_ref, asrc_ref, adst_ref, hh_ref, as_ref, ad_ref):
    hh = jnp.dot(h_ref[...], w_ref[...], preferred_element_type=jnp.float32)
    hh_ref[...] = hh
    as_ref[...] = jnp.dot(hh, asrc_ref[...], preferred_element_type=jnp.float32)
    ad_ref[...] = jnp.dot(hh, adst_ref[...], preferred_element_type=jnp.float32)


def _alpha_kernel(ea_ref, we_ref, ae_ref, asg_ref, adg_ref, o_ref):
    ee = jnp.dot(ea_ref[...], we_ref[...], preferred_element_type=jnp.float32)
    a_edge = jnp.dot(ee, ae_ref[...], preferred_element_type=jnp.float32)
    a = asg_ref[...] + adg_ref[...] + a_edge
    o_ref[...] = jnp.exp(jnp.where(a >= 0, a, 0.2 * a))


def _msg_kernel(ex_ref, hh_ref, r_ref, o_ref):
    ex64 = jnp.dot(ex_ref[...], r_ref[...], preferred_element_type=jnp.float32)
    o_ref[...] = hh_ref[...] * ex64


def _post_kernel(unagg_ref, den_ref, r_ref, hres_ref, b_ref, g_ref, beta_ref,
                 o_ref):
    den64 = jnp.dot(den_ref[...], r_ref[...], preferred_element_type=jnp.float32)
    z = unagg_ref[...] / (den64 + 1e-16) + b_ref[...] + hres_ref[...]
    mu = jnp.mean(z, axis=-1, keepdims=True)
    zc = z - mu
    var = jnp.mean(zc * zc, axis=-1, keepdims=True)
    hn = zc * jax.lax.rsqrt(var + 1e-5) * g_ref[...] + beta_ref[...]
    o_ref[...] = jax.nn.relu(hn)


def _pool_kernel(oh_ref, h_ref, sum_ref, cnt_ref):
    @pl.when(pl.program_id(0) == 0)
    def _init():
        sum_ref[...] = jnp.zeros_like(sum_ref)
        cnt_ref[...] = jnp.zeros_like(cnt_ref)

    oh = oh_ref[...]
    h = h_ref[...]
    sum_ref[...] += jax.lax.dot_general(
        oh, h, (((0,), (0,)), ((), ())), preferred_element_type=jnp.float32)
    cnt_ref[...] += jax.lax.dot_general(
        oh, jnp.ones_like(h), (((0,), (0,)), ((), ())),
        preferred_element_type=jnp.float32)


def _head_kernel(sum_ref, cnt_ref, wc1_ref, bc1_ref, wc2_ref, bc2_ref,
                 wn1_ref, bn1_ref, wn2_ref, bn2_ref,
                 gemb_ref, scene_ref, nav_ref):
    gemb = sum_ref[...] / jnp.maximum(cnt_ref[...], 1.0)
    gemb_ref[...] = gemb
    hidc = jax.nn.relu(
        jnp.dot(gemb, wc1_ref[...], preferred_element_type=jnp.float32)
        + bc1_ref[...])
    scene_ref[...] = (
        jnp.dot(hidc, wc2_ref[...], preferred_element_type=jnp.float32)
        + bc2_ref[...])
    hidn = jax.nn.relu(
        jnp.dot(gemb, wn1_ref[...], preferred_element_type=jnp.float32)
        + bn1_ref[...])
    nav_ref[...] = jax.nn.sigmoid(
        jnp.dot(hidn, wn2_ref[...], preferred_element_type=jnp.float32)
        + bn2_ref[...])


def _full(shape):
    return pl.BlockSpec(shape, lambda *i: tuple(0 for _ in shape))


def kernel(x, edge_index, edge_attr, batch, Wp, bp,
           W0, Asrc0, Adst0, Aedge0, We0, b0, g0, beta0,
           W1, Asrc1, Adst1, Aedge1, We1, b1, g1, beta1,
           W2, Asrc2, Adst2, Aedge2, We2, b2, g2, beta2,
           Wc1, bc1, Wc2, bc2, Wn1, bn1, Wn2, bn2):
    n = x.shape[0]
    e = edge_attr.shape[0]
    hid = Wp.shape[1]
    heads, ch = Asrc0.shape

    src0 = edge_index[0]
    dst0 = edge_index[1]
    loop = jnp.arange(n, dtype=src0.dtype)
    src = jnp.concatenate([src0, loop])
    dst = jnp.concatenate([dst0, loop])
    ea = jnp.concatenate([
        edge_attr,
        jnp.broadcast_to(jnp.mean(edge_attr, axis=0, keepdims=True),
                         (n, edge_attr.shape[1]))], axis=0)
    ee_rows = e + n

    gn = pl.cdiv(n, _NBLK)
    ge = pl.cdiv(ee_rows, _EBLK)

    rmat = jnp.repeat(jnp.eye(heads, dtype=jnp.float32), ch, axis=1)

    h = pl.pallas_call(
        _proj_kernel,
        grid=(gn,),
        in_specs=[pl.BlockSpec((_NBLK, x.shape[1]), lambda i: (i, 0)),
                  _full(Wp.shape),
                  _full((1, hid))],
        out_specs=pl.BlockSpec((_NBLK, hid), lambda i: (i, 0)),
        out_shape=jax.ShapeDtypeStruct((n, hid), jnp.float32),
    )(x, Wp, bp.reshape(1, hid))

    layers = [
        (W0, Asrc0, Adst0, Aedge0, We0, b0, g0, beta0),
        (W1, Asrc1, Adst1, Aedge1, We1, b1, g1, beta1),
        (W2, Asrc2, Adst2, Aedge2, We2, b2, g2, beta2),
    ]

    for (W, Asrc, Adst, Aedge, We, b, g, beta) in layers:
        hres = h
        hh, a_src, a_dst = pl.pallas_call(
            _node_mm_kernel,
            grid=(gn,),
            in_specs=[pl.BlockSpec((_NBLK, hid), lambda i: (i, 0)),
                      _full(W.shape),
                      _full((hid, heads)),
                      _full((hid, heads))],
            out_specs=[pl.BlockSpec((_NBLK, hid), lambda i: (i, 0)),
                       pl.BlockSpec((_NBLK, heads), lambda i: (i, 0)),
                       pl.BlockSpec((_NBLK, heads), lambda i: (i, 0))],
            out_shape=[jax.ShapeDtypeStruct((n, hid), jnp.float32),
                       jax.ShapeDtypeStruct((n, heads), jnp.float32),
                       jax.ShapeDtypeStruct((n, heads), jnp.float32)],
        )(h, W, _expand_a(Asrc), _expand_a(Adst))

        asg = a_src[src]
        adg = a_dst[dst]

        ex = pl.pallas_call(
            _alpha_kernel,
            grid=(ge,),
            in_specs=[pl.BlockSpec((_EBLK, ea.shape[1]), lambda i: (i, 0)),
                      _full(We.shape),
                      _full((hid, heads)),
                      pl.BlockSpec((_EBLK, heads), lambda i: (i, 0)),
                      pl.BlockSpec((_EBLK, heads), lambda i: (i, 0))],
            out_specs=pl.BlockSpec((_EBLK, heads), lambda i: (i, 0)),
            out_shape=jax.ShapeDtypeStruct((ee_rows, heads), jnp.float32),
        )(ea, We, _expand_a(Aedge), asg, adg)

        den = jax.ops.segment_sum(ex, dst, num_segments=n)
        msg = pl.pallas_call(
            _msg_kernel,
            grid=(ge,),
            in_specs=[pl.BlockSpec((_EBLK, heads), lambda i: (i, 0)),
                      pl.BlockSpec((_EBLK, hid), lambda i: (i, 0)),
                      _full((heads, hid))],
            out_specs=pl.BlockSpec((_EBLK, hid), lambda i: (i, 0)),
            out_shape=jax.ShapeDtypeStruct((ee_rows, hid), jnp.float32),
        )(ex, hh[src], rmat)

        unagg = jax.ops.segment_sum(msg, dst, num_segments=n)

        h = pl.pallas_call(
            _post_kernel,
            grid=(gn,),
            in_specs=[pl.BlockSpec((_NBLK, hid), lambda i: (i, 0)),
                      pl.BlockSpec((_NBLK, heads), lambda i: (i, 0)),
                      _full((heads, hid)),
                      pl.BlockSpec((_NBLK, hid), lambda i: (i, 0)),
                      _full((1, hid)),
                      _full((1, hid)),
                      _full((1, hid))],
            out_specs=pl.BlockSpec((_NBLK, hid), lambda i: (i, 0)),
            out_shape=jax.ShapeDtypeStruct((n, hid), jnp.float32),
        )(unagg, den, rmat, hres, b.reshape(1, hid), g.reshape(1, hid),
          beta.reshape(1, hid))

    onehot = (batch[:, None] == jnp.arange(_NGRAPHS, dtype=batch.dtype)[None, :]
              ).astype(jnp.float32)
    sums, cnts = pl.pallas_call(
        _pool_kernel,
        grid=(gn,),
        in_specs=[pl.BlockSpec((_NBLK, _NGRAPHS), lambda i: (i, 0)),
                  pl.BlockSpec((_NBLK, hid), lambda i: (i, 0))],
        out_specs=[_full((_NGRAPHS, hid)), _full((_NGRAPHS, hid))],
        out_shape=[jax.ShapeDtypeStruct((_NGRAPHS, hid), jnp.float32),
                   jax.ShapeDtypeStruct((_NGRAPHS, hid), jnp.float32)],
    )(onehot, h)

    gemb, scene_logits, nav = pl.pallas_call(
        _head_kernel,
        in_specs=[_full((_NGRAPHS, hid)), _full((_NGRAPHS, hid)),
                  _full(Wc1.shape), _full((1, Wc1.shape[1])),
                  _full(Wc2.shape), _full((1, Wc2.shape[1])),
                  _full(Wn1.shape), _full((1, Wn1.shape[1])),
                  _full(Wn2.shape), _full((1, Wn2.shape[1]))],
        out_specs=[_full((_NGRAPHS, hid)),
                   _full((_NGRAPHS, Wc2.shape[1])),
                   _full((_NGRAPHS, Wn2.shape[1]))],
        out_shape=[jax.ShapeDtypeStruct((_NGRAPHS, hid), jnp.float32),
                   jax.ShapeDtypeStruct((_NGRAPHS, Wc2.shape[1]), jnp.float32),
                   jax.ShapeDtypeStruct((_NGRAPHS, Wn2.shape[1]), jnp.float32)],
    )(sums, cnts, Wc1, bc1.reshape(1, -1), Wc2, bc2.reshape(1, -1),
      Wn1, bn1.reshape(1, -1), Wn2, bn2.reshape(1, -1))

    return (h, scene_logits, nav, gemb)


# speedup vs baseline: 19.6200x; 1.3116x over previous
import jax
import jax.numpy as jnp
from jax.experimental import pallas as pl

_NBLK = 2000
_EBLK = 4000
_NGRAPHS = 16


def _expand_a(a):
    heads, ch = a.shape
    return (a[:, :, None] * jnp.eye(heads, dtype=a.dtype)[:, None, :]).reshape(
        heads * ch, heads)


def _proj_kernel(x_ref, wp_ref, bp_ref, o_ref):
    o_ref[...] = jax.nn.relu(
        jnp.dot(x_ref[...], wp_ref[...], preferred_element_type=jnp.float32)
        + bp_ref[...])


def _node_mm_kernel(h_ref, w_ref, asrc_ref, adst_ref, hhs_ref, ad_ref):
    hh = jnp.dot(h_ref[...], w_ref[...], preferred_element_type=jnp.float32)
    a_src = jnp.dot(hh, asrc_ref[...], preferred_element_type=jnp.float32)
    hhs_ref[...] = jnp.concatenate([hh, a_src], axis=1)
    ad_ref[...] = jnp.dot(hh, adst_ref[...], preferred_element_type=jnp.float32)


def _edge_kernel(ea_ref, we_ref, ae_ref, g_ref, adg_ref, r_ref, o_ref):
    ee = jnp.dot(ea_ref[...], we_ref[...], preferred_element_type=jnp.float32)
    a_edge = jnp.dot(ee, ae_ref[...], preferred_element_type=jnp.float32)
    g = g_ref[...]
    hh_g = g[:, :64]
    asg = g[:, 64:68]
    a = asg + adg_ref[...] + a_edge
    ex = jnp.exp(jnp.where(a >= 0, a, 0.2 * a))
    ex64 = jnp.dot(ex, r_ref[...], preferred_element_type=jnp.float32)
    o_ref[...] = jnp.concatenate([hh_g * ex64, ex], axis=1)


def _post_kernel(seg_ref, r_ref, hres_ref, b_ref, g_ref, beta_ref, o_ref):
    seg = seg_ref[...]
    unagg = seg[:, :64]
    den64 = jnp.dot(seg[:, 64:68], r_ref[...],
                    preferred_element_type=jnp.float32)
    z = unagg / (den64 + 1e-16) + b_ref[...] + hres_ref[...]
    mu = jnp.mean(z, axis=-1, keepdims=True)
    zc = z - mu
    var = jnp.mean(zc * zc, axis=-1, keepdims=True)
    hn = zc * jax.lax.rsqrt(var + 1e-5) * g_ref[...] + beta_ref[...]
    o_ref[...] = jax.nn.relu(hn)


def _pool_kernel(oh_ref, h_ref, sum_ref, cnt_ref):
    @pl.when(pl.program_id(0) == 0)
    def _init():
        sum_ref[...] = jnp.zeros_like(sum_ref)
        cnt_ref[...] = jnp.zeros_like(cnt_ref)

    oh = oh_ref[...]
    h = h_ref[...]
    sum_ref[...] += jax.lax.dot_general(
        oh, h, (((0,), (0,)), ((), ())), preferred_element_type=jnp.float32)
    cnt_ref[...] += jax.lax.dot_general(
        oh, jnp.ones_like(h), (((0,), (0,)), ((), ())),
        preferred_element_type=jnp.float32)


def _head_kernel(sum_ref, cnt_ref, wc1_ref, bc1_ref, wc2_ref, bc2_ref,
                 wn1_ref, bn1_ref, wn2_ref, bn2_ref,
                 gemb_ref, scene_ref, nav_ref):
    gemb = sum_ref[...] / jnp.maximum(cnt_ref[...], 1.0)
    gemb_ref[...] = gemb
    hidc = jax.nn.relu(
        jnp.dot(gemb, wc1_ref[...], preferred_element_type=jnp.float32)
        + bc1_ref[...])
    scene_ref[...] = (
        jnp.dot(hidc, wc2_ref[...], preferred_element_type=jnp.float32)
        + bc2_ref[...])
    hidn = jax.nn.relu(
        jnp.dot(gemb, wn1_ref[...], preferred_element_type=jnp.float32)
        + bn1_ref[...])
    nav_ref[...] = jax.nn.sigmoid(
        jnp.dot(hidn, wn2_ref[...], preferred_element_type=jnp.float32)
        + bn2_ref[...])


def _full(shape):
    return pl.BlockSpec(shape, lambda *i: tuple(0 for _ in shape))


def kernel(x, edge_index, edge_attr, batch, Wp, bp,
           W0, Asrc0, Adst0, Aedge0, We0, b0, g0, beta0,
           W1, Asrc1, Adst1, Aedge1, We1, b1, g1, beta1,
           W2, Asrc2, Adst2, Aedge2, We2, b2, g2, beta2,
           Wc1, bc1, Wc2, bc2, Wn1, bn1, Wn2, bn2):
    n = x.shape[0]
    e = edge_attr.shape[0]
    hid = Wp.shape[1]
    heads, ch = Asrc0.shape

    src0 = edge_index[0]
    dst0 = edge_index[1]
    loop = jnp.arange(n, dtype=src0.dtype)
    src = jnp.concatenate([src0, loop])
    dst = jnp.concatenate([dst0, loop])
    ea = jnp.concatenate([
        edge_attr,
        jnp.broadcast_to(jnp.mean(edge_attr, axis=0, keepdims=True),
                         (n, edge_attr.shape[1]))], axis=0)
    ee_rows = e + n

    gn = pl.cdiv(n, _NBLK)
    ge = pl.cdiv(ee_rows, _EBLK)

    rmat = jnp.repeat(jnp.eye(heads, dtype=jnp.float32), ch, axis=1)

    h = pl.pallas_call(
        _proj_kernel,
        grid=(gn,),
        in_specs=[pl.BlockSpec((_NBLK, x.shape[1]), lambda i: (i, 0)),
                  _full(Wp.shape),
                  _full((1, hid))],
        out_specs=pl.BlockSpec((_NBLK, hid), lambda i: (i, 0)),
        out_shape=jax.ShapeDtypeStruct((n, hid), jnp.float32),
    )(x, Wp, bp.reshape(1, hid))

    layers = [
        (W0, Asrc0, Adst0, Aedge0, We0, b0, g0, beta0),
        (W1, Asrc1, Adst1, Aedge1, We1, b1, g1, beta1),
        (W2, Asrc2, Adst2, Aedge2, We2, b2, g2, beta2),
    ]

    hcat = hid + heads
    for (W, Asrc, Adst, Aedge, We, b, g, beta) in layers:
        hres = h
        hhs, a_dst = pl.pallas_call(
            _node_mm_kernel,
            grid=(gn,),
            in_specs=[pl.BlockSpec((_NBLK, hid), lambda i: (i, 0)),
                      _full(W.shape),
                      _full((hid, heads)),
                      _full((hid, heads))],
            out_specs=[pl.BlockSpec((_NBLK, hcat), lambda i: (i, 0)),
                       pl.BlockSpec((_NBLK, heads), lambda i: (i, 0))],
            out_shape=[jax.ShapeDtypeStruct((n, hcat), jnp.float32),
                       jax.ShapeDtypeStruct((n, heads), jnp.float32)],
        )(h, W, _expand_a(Asrc), _expand_a(Adst))

        gsrc = hhs[src]
        adg = a_dst[dst]

        msg68 = pl.pallas_call(
            _edge_kernel,
            grid=(ge,),
            in_specs=[pl.BlockSpec((_EBLK, ea.shape[1]), lambda i: (i, 0)),
                      _full(We.shape),
                      _full((hid, heads)),
                      pl.BlockSpec((_EBLK, hcat), lambda i: (i, 0)),
                      pl.BlockSpec((_EBLK, heads), lambda i: (i, 0)),
                      _full((heads, hid))],
            out_specs=pl.BlockSpec((_EBLK, hcat), lambda i: (i, 0)),
            out_shape=jax.ShapeDtypeStruct((ee_rows, hcat), jnp.float32),
        )(ea, We, _expand_a(Aedge), gsrc, adg, rmat)

        seg = jax.ops.segment_sum(msg68, dst, num_segments=n)

        h = pl.pallas_call(
            _post_kernel,
            grid=(gn,),
            in_specs=[pl.BlockSpec((_NBLK, hcat), lambda i: (i, 0)),
                      _full((heads, hid)),
                      pl.BlockSpec((_NBLK, hid), lambda i: (i, 0)),
                      _full((1, hid)),
                      _full((1, hid)),
                      _full((1, hid))],
            out_specs=pl.BlockSpec((_NBLK, hid), lambda i: (i, 0)),
            out_shape=jax.ShapeDtypeStruct((n, hid), jnp.float32),
        )(seg, rmat, hres, b.reshape(1, hid), g.reshape(1, hid),
          beta.reshape(1, hid))

    onehot = (batch[:, None] == jnp.arange(_NGRAPHS, dtype=batch.dtype)[None, :]
              ).astype(jnp.float32)
    sums, cnts = pl.pallas_call(
        _pool_kernel,
        grid=(gn,),
        in_specs=[pl.BlockSpec((_NBLK, _NGRAPHS), lambda i: (i, 0)),
                  pl.BlockSpec((_NBLK, hid), lambda i: (i, 0))],
        out_specs=[_full((_NGRAPHS, hid)), _full((_NGRAPHS, hid))],
        out_shape=[jax.ShapeDtypeStruct((_NGRAPHS, hid), jnp.float32),
                   jax.ShapeDtypeStruct((_NGRAPHS, hid), jnp.float32)],
    )(onehot, h)

    gemb, scene_logits, nav = pl.pallas_call(
        _head_kernel,
        in_specs=[_full((_NGRAPHS, hid)), _full((_NGRAPHS, hid)),
                  _full(Wc1.shape), _full((1, Wc1.shape[1])),
                  _full(Wc2.shape), _full((1, Wc2.shape[1])),
                  _full(Wn1.shape), _full((1, Wn1.shape[1])),
                  _full(Wn2.shape), _full((1, Wn2.shape[1]))],
        out_specs=[_full((_NGRAPHS, hid)),
                   _full((_NGRAPHS, Wc2.shape[1])),
                   _full((_NGRAPHS, Wn2.shape[1]))],
        out_shape=[jax.ShapeDtypeStruct((_NGRAPHS, hid), jnp.float32),
                   jax.ShapeDtypeStruct((_NGRAPHS, Wc2.shape[1]), jnp.float32),
                   jax.ShapeDtypeStruct((_NGRAPHS, Wn2.shape[1]), jnp.float32)],
    )(sums, cnts, Wc1, bc1.reshape(1, -1), Wc2, bc2.reshape(1, -1),
      Wn1, bn1.reshape(1, -1), Wn2, bn2.reshape(1, -1))

    return (h, scene_logits, nav, gemb)
